# Initial kernel scaffold; baseline (speedup 1.0000x reference)
import functools

import jax
import jax.numpy as jnp
from jax import lax
from jax.experimental import pallas as pl
from jax.experimental.pallas import tpu as pltpu
from jax.experimental.pallas import tpu_sc as plsc

B = 1048576
RES = 512
L = 16
NC, NS = 2, 16
NW = NC * NS
QPW = B // NW
NB = 512
NBLOCKS = QPW // NB
IDX_ROWS = NB // 128


def _sc_fetch_body(uv_hbm, table_hbm, feat_hbm,
                   uv_v, idx0_v, idx1_v, fu_v, fv_v,
                   rows0_v, rows1_v, feat_v, sem):
    cid = lax.axis_index("c")
    sid = lax.axis_index("s")
    wid = sid * NC + cid
    wbase = wid * QPW
    lanes = lax.iota(jnp.int32, L)
    zeros = jnp.zeros((L,), jnp.int32)
    ones = jnp.full((L,), 1, jnp.int32)

    def block_body(bi, carry):
        base = wbase + bi * NB
        pltpu.sync_copy(uv_hbm.at[pl.ds(base, NB)], uv_v)

        def idx_body(ci, c2):
            q = ci * L + lanes
            x = plsc.load_gather(uv_v, [q, zeros])
            y = plsc.load_gather(uv_v, [q, ones])
            up = x * 511.0
            vp = y * (-511.0)
            ui = up.astype(jnp.int32)
            uf = jnp.where(up < ui.astype(jnp.float32), ui - 1, ui)
            fu = up - uf.astype(jnp.float32)
            vi = vp.astype(jnp.int32)
            vf = jnp.where(vp < vi.astype(jnp.float32), vi - 1, vi)
            fv = vp - vf.astype(jnp.float32)
            u0 = jnp.bitwise_and(uf, RES - 1)
            v0 = jnp.bitwise_and(vf, RES - 1)
            v1 = jnp.bitwise_and(vf + 1, RES - 1)
            row = ci // 8
            col = (ci % 8) * L
            idx0_v[row, pl.ds(col, L)] = v0 * RES + u0
            idx1_v[row, pl.ds(col, L)] = v1 * RES + u0
            fu_v[pl.ds(ci * L, L)] = fu
            fv_v[pl.ds(ci * L, L)] = fv
            return c2

        lax.fori_loop(0, NB // L, idx_body, 0)

        handles = []
        for j in range(IDX_ROWS):
            handles.append(pltpu.async_copy(
                table_hbm.at[idx0_v.at[j]],
                rows0_v.at[pl.ds(j * 128, 128)], sem))
            handles.append(pltpu.async_copy(
                table_hbm.at[idx1_v.at[j]],
                rows1_v.at[pl.ds(j * 128, 128)], sem))
        for h in handles:
            h.wait()

        def comb_body(ci, c2):
            q = ci * L + lanes
            fu = fu_v[pl.ds(ci * L, L)]
            fv = fv_v[pl.ds(ci * L, L)]
            for ch in range(8):
                cs = jnp.full((L,), ch, jnp.int32)
                cs8 = jnp.full((L,), ch + 8, jnp.int32)
                a = plsc.load_gather(rows0_v, [q, cs])
                b = plsc.load_gather(rows0_v, [q, cs8])
                c = plsc.load_gather(rows1_v, [q, cs])
                d = plsc.load_gather(rows1_v, [q, cs8])
                t0 = a + fu * (b - a)
                t1 = c + fu * (d - c)
                f = t0 + fv * (t1 - t0)
                plsc.store_scatter(feat_v, [q, cs], f)
            return c2

        lax.fori_loop(0, NB // L, comb_body, 0)
        pltpu.sync_copy(feat_v, feat_hbm.at[pl.ds(base, NB)])
        return carry

    lax.fori_loop(0, NBLOCKS, block_body, 0)


_sc_fetch = pl.kernel(
    _sc_fetch_body,
    out_type=jax.ShapeDtypeStruct((B, 8), jnp.float32),
    mesh=plsc.VectorSubcoreMesh(core_axis_name="c", subcore_axis_name="s"),
    scratch_types=[
        pltpu.VMEM((NB, 2), jnp.float32),
        pltpu.VMEM((IDX_ROWS, 128), jnp.int32),
        pltpu.VMEM((IDX_ROWS, 128), jnp.int32),
        pltpu.VMEM((NB,), jnp.float32),
        pltpu.VMEM((NB,), jnp.float32),
        pltpu.VMEM((NB, 16), jnp.float32),
        pltpu.VMEM((NB, 16), jnp.float32),
        pltpu.VMEM((NB, 8), jnp.float32),
        pltpu.SemaphoreType.DMA,
    ],
)


def _pair_table(tex):
    t = jnp.transpose(tex, (1, 2, 0))
    tn = jnp.concatenate([t[:, 1:], t[:, :1]], axis=1)
    return jnp.concatenate([t, tn], axis=-1).reshape(RES * RES, 16)


BLK = 32768


def _mlp1_body(f_ref, wo_ref, uv_ref, w0a, w0b, b0, w1, b1, w2, b2, out_ref):
    f = f_ref[...]
    wo = wo_ref[...]
    h = (jnp.dot(f, w0a[...], preferred_element_type=jnp.float32)
         + jnp.dot(wo, w0b[...], preferred_element_type=jnp.float32)
         + b0[...])
    h = jnp.where(h >= 0, h, 0.01 * h)
    h = jnp.dot(h, w1[...], preferred_element_type=jnp.float32) + b1[...]
    h = jnp.where(h >= 0, h, 0.01 * h)
    r = jnp.dot(h, w2[...], preferred_element_type=jnp.float32) + b2[...]
    s = jnp.sum(wo * wo, axis=1, keepdims=True)
    denom = jnp.sqrt(jnp.maximum(1.0 - s, 0.36))
    out_ref[...] = r / denom * wo + uv_ref[...]


def _mlp2_body(f_ref, wi_ref, wo_ref, w0a, w0b, w0c, b0, w1, b1, w2, b2,
               out_ref):
    h = (jnp.dot(wi_ref[...], w0a[...], preferred_element_type=jnp.float32)
         + jnp.dot(wo_ref[...], w0b[...], preferred_element_type=jnp.float32)
         + jnp.dot(f_ref[...], w0c[...], preferred_element_type=jnp.float32)
         + b0[...])
    h = jnp.where(h >= 0, h, 0.01 * h)
    h = jnp.dot(h, w1[...], preferred_element_type=jnp.float32) + b1[...]
    h = jnp.where(h >= 0, h, 0.01 * h)
    h = jnp.dot(h, w2[...], preferred_element_type=jnp.float32) + b2[...]
    out_ref[...] = jnp.maximum(h, 0.0)


def _data_spec(ncols):
    return pl.BlockSpec((BLK, ncols), lambda i: (i, 0))


def _w_spec(shape):
    return pl.BlockSpec(shape, lambda i: (0, 0))


def kernel(uv, wo, wi, offset_texture, rgb_texture,
           off_W0, off_b0, off_W1, off_b1, off_W2, off_b2,
           rgb_W0, rgb_b0, rgb_W1, rgb_b1, rgb_W2, rgb_b2):
    table_off = _pair_table(offset_texture)
    table_rgb = _pair_table(rgb_texture)

    f_off = _sc_fetch(uv, table_off)

    grid = (B // BLK,)
    uv_new = pl.pallas_call(
        _mlp1_body,
        grid=grid,
        in_specs=[_data_spec(8), _data_spec(2), _data_spec(2),
                  _w_spec((8, 32)), _w_spec((2, 32)), _w_spec((1, 32)),
                  _w_spec((32, 32)), _w_spec((1, 32)),
                  _w_spec((32, 1)), _w_spec((1, 1))],
        out_specs=_data_spec(2),
        out_shape=jax.ShapeDtypeStruct((B, 2), jnp.float32),
    )(f_off, wo, uv, off_W0[:8], off_W0[8:], off_b0.reshape(1, 32),
      off_W1, off_b1.reshape(1, 32), off_W2, off_b2.reshape(1, 1))

    f_rgb = _sc_fetch(uv_new, table_rgb)

    out = pl.pallas_call(
        _mlp2_body,
        grid=grid,
        in_specs=[_data_spec(8), _data_spec(2), _data_spec(2),
                  _w_spec((2, 32)), _w_spec((2, 32)), _w_spec((8, 32)),
                  _w_spec((1, 32)), _w_spec((32, 32)), _w_spec((1, 32)),
                  _w_spec((32, 3)), _w_spec((1, 3))],
        out_specs=_data_spec(3),
        out_shape=jax.ShapeDtypeStruct((B, 3), jnp.float32),
    )(f_rgb, wi, wo, rgb_W0[:2], rgb_W0[2:4], rgb_W0[4:],
      rgb_b0.reshape(1, 32), rgb_W1, rgb_b1.reshape(1, 32),
      rgb_W2, rgb_b2.reshape(1, 3))

    return out


# speedup vs baseline: 28.1835x; 28.1835x over previous
import functools

import jax
import jax.numpy as jnp
from jax import lax
from jax.experimental import pallas as pl
from jax.experimental.pallas import tpu as pltpu
from jax.experimental.pallas import tpu_sc as plsc

B = 1048576
RES = 512
L = 16
NC, NS = 2, 16
NW = NC * NS
QPW = B // NW
NB = 256
NBLOCKS = QPW // NB

_GDN = lax.GatherDimensionNumbers(offset_dims=(), collapsed_slice_dims=(0,),
                                  start_index_map=(0,))


def _perm(v, idx):
    return lax.gather(v, idx[:, None], _GDN, (1,),
                      mode=lax.GatherScatterMode.PROMISE_IN_BOUNDS)


def _sc_fetch_body(ux_hbm, vy_hbm, table_hbm, feat_hbm,
                   ux_v, vy_v, idx_v, off_v, fu_v, fv_v,
                   rows_v, feat_v, sem):
    cid = lax.axis_index("c")
    sid = lax.axis_index("s")
    wid = sid * NC + cid
    wbase = wid * QPW

    def block_body(bi, carry):
        base = wbase + bi * NB
        pltpu.sync_copy(ux_hbm.at[pl.ds(base, NB)], ux_v)
        pltpu.sync_copy(vy_hbm.at[pl.ds(base, NB)], vy_v)

        def idx_body(ci, c2):
            x = ux_v[pl.ds(ci * L, L)]
            y = vy_v[pl.ds(ci * L, L)]
            up = x * 511.0
            vp = y * (-511.0)
            ui = up.astype(jnp.int32)
            uf = jnp.where(up < ui.astype(jnp.float32), ui - 1, ui)
            fu = up - uf.astype(jnp.float32)
            vi = vp.astype(jnp.int32)
            vf = jnp.where(vp < vi.astype(jnp.float32), vi - 1, vi)
            fv = vp - vf.astype(jnp.float32)
            u0 = jnp.bitwise_and(uf, RES - 1)
            v0 = jnp.bitwise_and(vf, RES - 1)
            idx_v[pl.ds(ci * L, L)] = v0 * (RES // 4) + (u0 >> 2)
            off_v[pl.ds(ci * L, L)] = jnp.bitwise_and(u0, 3) * 32
            fu_v[pl.ds(ci * L, L)] = fu
            fv_v[pl.ds(ci * L, L)] = fv
            return c2

        lax.fori_loop(0, NB // L, idx_body, 0)

        handles = []
        for j in range(NB // 128):
            handles.append(pltpu.async_copy(
                table_hbm.at[idx_v.at[pl.ds(j * 128, 128)]],
                rows_v.at[pl.ds(j * 128, 128)], sem))
        for h in handles:
            h.wait()

        lanes = lax.iota(jnp.int32, L)
        fold_idx = jnp.bitwise_and(lanes, 7) + 8

        def comb_body(ci, c2):
            qb = ci * L
            fuv = fu_v[pl.ds(qb, L)]
            fvv = fv_v[pl.ds(qb, L)]
            offv = off_v[pl.ds(qb, L)]
            for j in range(L):
                q = qb + j
                off = offv[j]
                bj = jnp.full((L,), j, jnp.int32)
                fus = _perm(fuv, bj)
                fvs = _perm(fvv, bj)
                x = rows_v[q, pl.ds(off, L)]
                y = rows_v[q, pl.ds(off + L, L)]
                p = x + fus * (y - x)
                t1 = _perm(p, fold_idx)
                f = p + fvs * (t1 - p)
                feat_v[pl.ds(q * 8, L)] = f
            return c2

        lax.fori_loop(0, NB // L, comb_body, 0)
        pltpu.sync_copy(feat_v.at[pl.ds(0, NB * 8)],
                        feat_hbm.at[pl.ds(base * 8, NB * 8)])
        return carry

    lax.fori_loop(0, NBLOCKS, block_body, 0)


_sc_fetch = pl.kernel(
    _sc_fetch_body,
    out_type=jax.ShapeDtypeStruct((B * 8,), jnp.float32),
    mesh=plsc.VectorSubcoreMesh(core_axis_name="c", subcore_axis_name="s"),
    scratch_types=[
        pltpu.VMEM((NB,), jnp.float32),
        pltpu.VMEM((NB,), jnp.float32),
        pltpu.VMEM((NB,), jnp.int32),
        pltpu.VMEM((NB,), jnp.int32),
        pltpu.VMEM((NB,), jnp.float32),
        pltpu.VMEM((NB,), jnp.float32),
        pltpu.VMEM((NB, 128), jnp.float32),
        pltpu.VMEM((NB * 8 + 8,), jnp.float32),
        pltpu.SemaphoreType.DMA,
    ],
)


def _quad_table(tex):
    t = jnp.transpose(tex, (1, 2, 0))
    tv = jnp.concatenate([t[1:], t[:1]], axis=0)
    tu = jnp.concatenate([t[:, 1:], t[:, :1]], axis=1)
    tvu = jnp.concatenate([tv[:, 1:], tv[:, :1]], axis=1)
    quad = jnp.concatenate([t, tv, tu, tvu], axis=-1)
    return quad.reshape(RES * (RES // 4), 128)


BLK = 8192


def _mlp1_body(f_ref, wo_ref, uv_ref, w0a, w0b, b0, w1, b1, w2, b2, out_ref):
    f = f_ref[...]
    wo = wo_ref[...]
    h = (jnp.dot(f, w0a[...], preferred_element_type=jnp.float32)
         + jnp.dot(wo, w0b[...], preferred_element_type=jnp.float32)
         + b0[...])
    h = jnp.where(h >= 0, h, 0.01 * h)
    h = jnp.dot(h, w1[...], preferred_element_type=jnp.float32) + b1[...]
    h = jnp.where(h >= 0, h, 0.01 * h)
    r = jnp.dot(h, w2[...], preferred_element_type=jnp.float32) + b2[...]
    s = jnp.sum(wo * wo, axis=1, keepdims=True)
    denom = jnp.sqrt(jnp.maximum(1.0 - s, 0.36))
    out_ref[...] = r / denom * wo + uv_ref[...]


def _mlp2_body(f_ref, wi_ref, wo_ref, w0a, w0b, w0c, b0, w1, b1, w2, b2,
               out_ref):
    h = (jnp.dot(wi_ref[...], w0a[...], preferred_element_type=jnp.float32)
         + jnp.dot(wo_ref[...], w0b[...], preferred_element_type=jnp.float32)
         + jnp.dot(f_ref[...], w0c[...], preferred_element_type=jnp.float32)
         + b0[...])
    h = jnp.where(h >= 0, h, 0.01 * h)
    h = jnp.dot(h, w1[...], preferred_element_type=jnp.float32) + b1[...]
    h = jnp.where(h >= 0, h, 0.01 * h)
    h = jnp.dot(h, w2[...], preferred_element_type=jnp.float32) + b2[...]
    out_ref[...] = jnp.maximum(h, 0.0)


def _data_spec(ncols):
    return pl.BlockSpec((BLK, ncols), lambda i: (i, 0))


def _w_spec(shape):
    return pl.BlockSpec(shape, lambda i: (0, 0))


def kernel(uv, wo, wi, offset_texture, rgb_texture,
           off_W0, off_b0, off_W1, off_b1, off_W2, off_b2,
           rgb_W0, rgb_b0, rgb_W1, rgb_b1, rgb_W2, rgb_b2):
    table_off = _quad_table(offset_texture)
    table_rgb = _quad_table(rgb_texture)

    f_off = _sc_fetch(uv[:, 0], uv[:, 1], table_off).reshape(B, 8)

    grid = (B // BLK,)
    uv_new = pl.pallas_call(
        _mlp1_body,
        grid=grid,
        in_specs=[_data_spec(8), _data_spec(2), _data_spec(2),
                  _w_spec((8, 32)), _w_spec((2, 32)), _w_spec((1, 32)),
                  _w_spec((32, 32)), _w_spec((1, 32)),
                  _w_spec((32, 1)), _w_spec((1, 1))],
        out_specs=_data_spec(2),
        out_shape=jax.ShapeDtypeStruct((B, 2), jnp.float32),
    )(f_off, wo, uv, off_W0[:8], off_W0[8:], off_b0.reshape(1, 32),
      off_W1, off_b1.reshape(1, 32), off_W2, off_b2.reshape(1, 1))

    f_rgb = _sc_fetch(uv_new[:, 0], uv_new[:, 1], table_rgb).reshape(B, 8)

    out = pl.pallas_call(
        _mlp2_body,
        grid=grid,
        in_specs=[_data_spec(8), _data_spec(2), _data_spec(2),
                  _w_spec((2, 32)), _w_spec((2, 32)), _w_spec((8, 32)),
                  _w_spec((1, 32)), _w_spec((32, 32)), _w_spec((1, 32)),
                  _w_spec((32, 3)), _w_spec((1, 3))],
        out_specs=_data_spec(3),
        out_shape=jax.ShapeDtypeStruct((B, 3), jnp.float32),
    )(f_rgb, wi, wo, rgb_W0[:2], rgb_W0[2:4], rgb_W0[4:],
      rgb_b0.reshape(1, 32), rgb_W1, rgb_b1.reshape(1, 32),
      rgb_W2, rgb_b2.reshape(1, 3))

    return out


# speedup vs baseline: 40.0023x; 1.4194x over previous
import jax
import jax.numpy as jnp
from jax import lax
from jax.experimental import pallas as pl
from jax.experimental.pallas import tpu as pltpu
from jax.experimental.pallas import tpu_sc as plsc

B = 1048576
RES = 512
L = 16
NC, NS = 2, 16
NW = NC * NS
QPW = B // NW
NB = 256
NBLOCKS = QPW // NB

_GDN = lax.GatherDimensionNumbers(offset_dims=(), collapsed_slice_dims=(0,),
                                  start_index_map=(0,))


def _perm(v, idx):
    return lax.gather(v, idx[:, None], _GDN, (1,),
                      mode=lax.GatherScatterMode.PROMISE_IN_BOUNDS)


def _sc_fetch_body(uv_hbm, off_hbm, table_hbm, feat_hbm,
                   uv_v, ox_v, oy_v, idx_v, off_v, fu_v, fv_v,
                   rows_v, featT_v, sem):
    cid = lax.axis_index("c")
    sid = lax.axis_index("s")
    wid = sid * NC + cid
    wbase = wid * QPW
    lanes = lax.iota(jnp.int32, L)
    ieven = jnp.bitwise_and(2 * lanes, 15)
    iodd = ieven + 1

    def block_body(bi, carry):
        base = wbase + bi * NB
        pltpu.sync_copy(uv_hbm.at[pl.ds(2 * base, 2 * NB)], uv_v)
        if off_hbm is not None:
            pltpu.sync_copy(off_hbm.at[0, pl.ds(base, NB)], ox_v)
            pltpu.sync_copy(off_hbm.at[1, pl.ds(base, NB)], oy_v)

        def idx_body(ci, c2):
            a = uv_v[pl.ds(ci * 2 * L, L)]
            b = uv_v[pl.ds(ci * 2 * L + L, L)]
            x = jnp.where(lanes < 8, _perm(a, ieven), _perm(b, ieven))
            y = jnp.where(lanes < 8, _perm(a, iodd), _perm(b, iodd))
            if off_hbm is not None:
                x = x + ox_v[pl.ds(ci * L, L)]
                y = y + oy_v[pl.ds(ci * L, L)]
            up = x * 511.0
            vp = y * (-511.0)
            ui = up.astype(jnp.int32)
            uf = jnp.where(up < ui.astype(jnp.float32), ui - 1, ui)
            fu = up - uf.astype(jnp.float32)
            vi = vp.astype(jnp.int32)
            vf = jnp.where(vp < vi.astype(jnp.float32), vi - 1, vi)
            fv = vp - vf.astype(jnp.float32)
            u0 = jnp.bitwise_and(uf, RES - 1)
            v0 = jnp.bitwise_and(vf, RES - 1)
            idx_v[pl.ds(ci * L, L)] = v0 * (RES // 4) + (u0 >> 2)
            off_v[pl.ds(ci * L, L)] = jnp.bitwise_and(u0, 3) * 32
            fu_v[pl.ds(ci * L, L)] = fu
            fv_v[pl.ds(ci * L, L)] = fv
            return c2

        lax.fori_loop(0, NB // L, idx_body, 0)

        handles = []
        for j in range(NB // 128):
            handles.append(pltpu.async_copy(
                table_hbm.at[idx_v.at[pl.ds(j * 128, 128)]],
                rows_v.at[pl.ds(j * 128, 128)], sem))
        for h in handles:
            h.wait()

        fold_idx = jnp.bitwise_and(lanes, 7) + 8
        xor_idx = {s: jnp.bitwise_xor(lanes, s) for s in (8, 4, 2, 1)}
        low_msk = {s: jnp.bitwise_and(lanes, s) == 0 for s in (8, 4, 2, 1)}

        def comb_body(ci, c2):
            qb = ci * L
            fuv = fu_v[pl.ds(qb, L)]
            fvv = fv_v[pl.ds(qb, L)]
            offv = off_v[pl.ds(qb, L)]
            fs = []
            for j in range(L):
                q = qb + j
                off = offv[j]
                bj = jnp.full((L,), j, jnp.int32)
                fus = _perm(fuv, bj)
                fvs = _perm(fvv, bj)
                x = rows_v[q, pl.ds(off, L)]
                y = rows_v[q, pl.ds(off + L, L)]
                p = x + fus * (y - x)
                t1 = _perm(p, fold_idx)
                fs.append(p + fvs * (t1 - p))
            r = [jnp.where(low_msk[8], fs[i], _perm(fs[i + 8], xor_idx[8]))
                 for i in range(8)]
            for s in (4, 2, 1):
                nr = list(r)
                for i in range(8):
                    if i & s:
                        continue
                    a, b = r[i], r[i + s]
                    nr[i] = jnp.where(low_msk[s], a, _perm(b, xor_idx[s]))
                    nr[i + s] = jnp.where(low_msk[s], _perm(a, xor_idx[s]), b)
                r = nr
            for c in range(8):
                featT_v[c, pl.ds(qb, L)] = r[c]
            return c2

        lax.fori_loop(0, NB // L, comb_body, 0)
        pltpu.sync_copy(featT_v, feat_hbm.at[:, pl.ds(base, NB)])
        return carry

    lax.fori_loop(0, NBLOCKS, block_body, 0)


def _sc_a_body(uv_hbm, table_hbm, feat_hbm,
               uv_v, idx_v, off_v, fu_v, fv_v, rows_v, featT_v, sem):
    _sc_fetch_body(uv_hbm, None, table_hbm, feat_hbm,
                   uv_v, None, None, idx_v, off_v, fu_v, fv_v,
                   rows_v, featT_v, sem)


_SCRATCH_A = [
    pltpu.VMEM((2 * NB,), jnp.float32),
    pltpu.VMEM((NB,), jnp.int32),
    pltpu.VMEM((NB,), jnp.int32),
    pltpu.VMEM((NB,), jnp.float32),
    pltpu.VMEM((NB,), jnp.float32),
    pltpu.VMEM((NB, 128), jnp.float32),
    pltpu.VMEM((8, NB), jnp.float32),
    pltpu.SemaphoreType.DMA,
]

_SCRATCH_B = ([pltpu.VMEM((2 * NB,), jnp.float32),
               pltpu.VMEM((NB,), jnp.float32),
               pltpu.VMEM((NB,), jnp.float32)] + _SCRATCH_A[1:])

_MESH = plsc.VectorSubcoreMesh(core_axis_name="c", subcore_axis_name="s")

_sc_fetch_a = pl.kernel(
    _sc_a_body,
    out_type=jax.ShapeDtypeStruct((8, B), jnp.float32),
    mesh=_MESH,
    scratch_types=_SCRATCH_A,
)

_sc_fetch_b = pl.kernel(
    _sc_fetch_body,
    out_type=jax.ShapeDtypeStruct((8, B), jnp.float32),
    mesh=_MESH,
    scratch_types=_SCRATCH_B,
)


def _quad_table(tex):
    t = jnp.transpose(tex, (1, 2, 0))
    tv = jnp.concatenate([t[1:], t[:1]], axis=0)
    tu = jnp.concatenate([t[:, 1:], t[:, :1]], axis=1)
    tvu = jnp.concatenate([tv[:, 1:], tv[:, :1]], axis=1)
    quad = jnp.concatenate([t, tv, tu, tvu], axis=-1)
    return quad.reshape(RES * (RES // 4), 128)


BLKC = 65536
BLKC2 = 16384


def _mlp1_body(f_ref, wo_ref, w0a, w0b, b0, w1, b1, w2, b2, out_ref):
    f = f_ref[...]
    wo = wo_ref[...]
    h = (jnp.dot(w0a[...], f, preferred_element_type=jnp.float32)
         + jnp.dot(w0b[...], wo, preferred_element_type=jnp.float32)
         + b0[...])
    h = jnp.where(h >= 0, h, 0.01 * h)
    h = jnp.dot(w1[...], h, preferred_element_type=jnp.float32) + b1[...]
    h = jnp.where(h >= 0, h, 0.01 * h)
    r = jnp.dot(w2[...], h, preferred_element_type=jnp.float32) + b2[...]
    s = jnp.sum(wo * wo, axis=0, keepdims=True)
    denom = jnp.sqrt(jnp.maximum(1.0 - s, 0.36))
    out_ref[...] = r / denom * wo


def _mlp2_body(f_ref, wi_ref, wo_ref, w0a, w0b, w0c, b0, w1, b1, w2, b2,
               out_ref):
    h = (jnp.dot(w0a[...], wi_ref[...], preferred_element_type=jnp.float32)
         + jnp.dot(w0b[...], wo_ref[...], preferred_element_type=jnp.float32)
         + jnp.dot(w0c[...], f_ref[...], preferred_element_type=jnp.float32)
         + b0[...])
    h = jnp.where(h >= 0, h, 0.01 * h)
    h = jnp.dot(w1[...], h, preferred_element_type=jnp.float32) + b1[...]
    h = jnp.where(h >= 0, h, 0.01 * h)
    h = jnp.dot(w2[...], h, preferred_element_type=jnp.float32) + b2[...]
    out_ref[...] = jnp.maximum(h, 0.0).T


def _wide_spec(nrows, blk):
    return pl.BlockSpec((nrows, blk), lambda i: (0, i))


def _w_spec(shape):
    return pl.BlockSpec(shape, lambda i: (0, 0))


def kernel(uv, wo, wi, offset_texture, rgb_texture,
           off_W0, off_b0, off_W1, off_b1, off_W2, off_b2,
           rgb_W0, rgb_b0, rgb_W1, rgb_b1, rgb_W2, rgb_b2):
    uvflat = uv.reshape(2 * B)
    woT = wo.T
    wiT = wi.T
    table_off = _quad_table(offset_texture)
    table_rgb = _quad_table(rgb_texture)

    f_offT = _sc_fetch_a(uvflat, table_off)

    uv_offT = pl.pallas_call(
        _mlp1_body,
        grid=(B // BLKC,),
        in_specs=[_wide_spec(8, BLKC), _wide_spec(2, BLKC),
                  _w_spec((32, 8)), _w_spec((32, 2)), _w_spec((32, 1)),
                  _w_spec((32, 32)), _w_spec((32, 1)),
                  _w_spec((1, 32)), _w_spec((1, 1))],
        out_specs=_wide_spec(2, BLKC),
        out_shape=jax.ShapeDtypeStruct((2, B), jnp.float32),
    )(f_offT, woT, off_W0[:8].T, off_W0[8:].T, off_b0.reshape(32, 1),
      off_W1.T, off_b1.reshape(32, 1), off_W2.T, off_b2.reshape(1, 1))

    f_rgbT = _sc_fetch_b(uvflat, uv_offT, table_rgb)

    out = pl.pallas_call(
        _mlp2_body,
        grid=(B // BLKC2,),
        in_specs=[_wide_spec(8, BLKC2), _wide_spec(2, BLKC2),
                  _wide_spec(2, BLKC2),
                  _w_spec((32, 2)), _w_spec((32, 2)), _w_spec((32, 8)),
                  _w_spec((32, 1)), _w_spec((32, 32)), _w_spec((32, 1)),
                  _w_spec((3, 32)), _w_spec((3, 1))],
        out_specs=pl.BlockSpec((BLKC2, 3), lambda i: (i, 0)),
        out_shape=jax.ShapeDtypeStruct((B, 3), jnp.float32),
    )(f_rgbT, wiT, woT, rgb_W0[:2].T, rgb_W0[2:4].T, rgb_W0[4:].T,
      rgb_b0.reshape(32, 1), rgb_W1.T, rgb_b1.reshape(32, 1),
      rgb_W2.T, rgb_b2.reshape(3, 1))

    return out
